# Initial kernel scaffold; baseline (speedup 1.0000x reference)
import jax
import jax.numpy as jnp
from jax.experimental import pallas as pl


def kernel(source, tok_emb, proj_in, Wqk, Wv, Wo, ln1_g, ln1_b, ln2_g, ln2_b, Wff1, bff1, Wff2, bff2, lnf_g, lnf_b, Wout, bout):
    raise NotImplementedError("write your pallas kernel here")


# speedup vs baseline: 2.2613x; 2.2613x over previous
import functools

import numpy as np
import jax
import jax.numpy as jnp
from jax import lax
from jax.experimental import pallas as pl
from jax.experimental.pallas import tpu as pltpu
from jax.experimental.pallas import tpu_sc as plsc

S = 4096
EMB = 128
DIM = 512
HEADS = 8
DH = 64
BUCKET = 64
NB = 64
NCH = S // BUCKET

_NC, _NS = 2, 16
_NW = _NC * _NS
_SUB = 512


def _pos_emb():
    pos = np.arange(S)[:, None].astype(np.float64)
    i = np.arange(EMB)[None, :].astype(np.float64)
    angle = pos / np.power(10000.0, (2.0 * (i // 2)) / EMB)
    pe = np.zeros((S, EMB))
    pe[:, 0::2] = np.sin(angle[:, 0::2])
    pe[:, 1::2] = np.cos(angle[:, 1::2])
    return jnp.asarray(pe, dtype=jnp.float32)


def _rot():
    return jnp.asarray(
        np.random.default_rng(42).standard_normal((HEADS, DH, NB // 2)),
        dtype=jnp.float32)


def _sc_mesh():
    return plsc.VectorSubcoreMesh(core_axis_name="c", subcore_axis_name="s",
                                  num_cores=_NC, num_subcores=_NS)


def _sc_gather_rows(table, idx):
    B = idx.shape[0]
    D = table.shape[1]
    bpw = B // _NW
    sub = min(bpw, _SUB)
    nst = bpw // sub
    kc = sub // 128
    idx3 = idx.reshape(_NW, bpw // 128, 128)

    @functools.partial(
        pl.kernel,
        out_type=jax.ShapeDtypeStruct((B, D), jnp.float32),
        mesh=_sc_mesh(),
        scratch_types=[
            pltpu.VMEM((bpw // 128, 128), jnp.int32),
            pltpu.VMEM((sub, D), jnp.float32),
            pltpu.SemaphoreType.DMA,
        ],
    )
    def k(table_hbm, idx_hbm, out_hbm, idx_v, rows_v, sem):
        wid = lax.axis_index("s") * _NC + lax.axis_index("c")
        base = wid * bpw
        pltpu.sync_copy(idx_hbm.at[wid], idx_v)
        for t in range(nst):
            cps = [
                pltpu.async_copy(table_hbm.at[idx_v.at[t * kc + j]],
                                 rows_v.at[pl.ds(j * 128, 128)], sem)
                for j in range(kc)
            ]
            for c in cps:
                c.wait()
            pltpu.sync_copy(rows_v, out_hbm.at[pl.ds(base + t * sub, sub)])

    return k(table, idx3)


def _sc_scatter2(d1, d2, idx):
    B, D = d1.shape
    bpw = B // _NW
    sub = min(bpw, _SUB)
    nst = bpw // sub
    kc = sub // 128
    idx3 = idx.reshape(_NW, bpw // 128, 128)

    @functools.partial(
        pl.kernel,
        out_type=[
            jax.ShapeDtypeStruct((B, D), jnp.float32),
            jax.ShapeDtypeStruct((B, D), jnp.float32),
        ],
        mesh=_sc_mesh(),
        scratch_types=[
            pltpu.VMEM((bpw // 128, 128), jnp.int32),
            pltpu.VMEM((sub, D), jnp.float32),
            pltpu.SemaphoreType.DMA,
        ],
    )
    def k(d1_hbm, d2_hbm, idx_hbm, o1_hbm, o2_hbm, idx_v, buf, sem):
        wid = lax.axis_index("s") * _NC + lax.axis_index("c")
        base = wid * bpw
        pltpu.sync_copy(idx_hbm.at[wid], idx_v)
        for src, dst in ((d1_hbm, o1_hbm), (d2_hbm, o2_hbm)):
            for t in range(nst):
                pltpu.sync_copy(src.at[pl.ds(base + t * sub, sub)], buf)
                cps = [
                    pltpu.async_copy(buf.at[pl.ds(j * 128, 128)],
                                     dst.at[idx_v.at[t * kc + j]], sem)
                    for j in range(kc)
                ]
                for c in cps:
                    c.wait()

    return k(d1, d2, idx3)


def _tc_hash(qk3, rot):
    def body(q_ref, rot_ref, o_ref):
        o_ref[0] = jnp.dot(q_ref[0], rot_ref[0],
                           preferred_element_type=jnp.float32)

    return pl.pallas_call(
        body,
        grid=(HEADS,),
        in_specs=[
            pl.BlockSpec((1, S, DH), lambda h: (h, 0, 0)),
            pl.BlockSpec((1, DH, NB // 2), lambda h: (h, 0, 0)),
        ],
        out_specs=pl.BlockSpec((1, S, NB // 2), lambda h: (h, 0, 0)),
        out_shape=jax.ShapeDtypeStruct((HEADS, S, NB // 2), jnp.float32),
    )(qk3, rot)


def _tc_rank(r):
    CH = 512

    def body(r_ref, o_ref):
        h = pl.program_id(0)
        rh = r_ref[0]
        full = jnp.concatenate([rh, -rh], axis=1)
        m = jnp.max(full, axis=1, keepdims=True)
        lane = lax.broadcasted_iota(jnp.int32, (S, NB), 1)
        first = jnp.min(jnp.where(full >= m, lane, NB),
                        axis=1, keepdims=True)
        oh = (lane == first).astype(jnp.float32)
        ltri = (lax.broadcasted_iota(jnp.int32, (CH, CH), 0) >
                lax.broadcasted_iota(jnp.int32, (CH, CH), 1)
                ).astype(jnp.float32)
        parts = []
        run = jnp.zeros((1, NB), jnp.float32)
        for i in range(S // CH):
            c = oh[i * CH:(i + 1) * CH]
            parts.append(jnp.dot(ltri, c, preferred_element_type=jnp.float32,
                                 precision=lax.Precision.HIGHEST) + run)
            run = run + jnp.sum(c, axis=0, keepdims=True)
        within = jnp.concatenate(parts, axis=0)
        utri = (lax.broadcasted_iota(jnp.int32, (NB, NB), 0) <
                lax.broadcasted_iota(jnp.int32, (NB, NB), 1)
                ).astype(jnp.float32)
        offs = jnp.dot(run, utri, preferred_element_type=jnp.float32,
                       precision=lax.Precision.HIGHEST)
        rank = jnp.sum(oh * (within + offs), axis=1, keepdims=True)
        o_ref[...] = (rank + h * S).astype(jnp.int32)[None]

    return pl.pallas_call(
        body,
        grid=(HEADS,),
        in_specs=[pl.BlockSpec((1, S, NB // 2), lambda h: (h, 0, 0))],
        out_specs=pl.BlockSpec((1, S, 1), lambda h: (h, 0, 0)),
        out_shape=jax.ShapeDtypeStruct((HEADS, S, 1), jnp.int32),
    )(r)


def _tc_attn(sqkv, nrm3, stick):
    def body(c_ref, p_ref, nc_ref, np_ref, tc_ref, tp_ref, o_ref):
        cur = c_ref[0]
        prv = p_ref[0]
        bq = cur[:, :DH]
        kw = jnp.concatenate([prv[:, :DH], cur[:, :DH]], axis=0)
        vw = jnp.concatenate([prv[:, DH:], cur[:, DH:]], axis=0)
        nrm = jnp.concatenate([np_ref[0], nc_ref[0]], axis=0)
        bk = kw / (nrm + 1e-6)
        dots = lax.dot_general(bq, bk, (((1,), (1,)), ((), ())),
                               preferred_element_type=jnp.float32) / 8.0
        tq = tc_ref[0][:, 0:1]
        tkc = jnp.concatenate([tp_ref[0], tc_ref[0]], axis=0)
        ones = jnp.full((1, 128), 1.0 / 128.0, jnp.float32)
        tk = lax.dot_general(ones, tkc, (((1,), (1,)), ((), ())),
                             preferred_element_type=jnp.float32,
                             precision=lax.Precision.HIGHEST)
        dots = jnp.where(tq < tk, -1e9, dots)
        dots = jnp.where(tq == tk, dots - 1e5, dots)
        attn = jax.nn.softmax(dots, axis=-1)
        out = jnp.dot(attn, vw, preferred_element_type=jnp.float32)
        o_ref[0] = jnp.concatenate(
            [out, jnp.zeros((BUCKET, DH), jnp.float32)], axis=1)

    prev = lambda h, c: (h, (c + NCH - 1) % NCH, 0)
    cur = lambda h, c: (h, c, 0)
    return pl.pallas_call(
        body,
        grid=(HEADS, NCH),
        in_specs=[
            pl.BlockSpec((1, BUCKET, 2 * DH), cur),
            pl.BlockSpec((1, BUCKET, 2 * DH), prev),
            pl.BlockSpec((1, BUCKET, 1), cur),
            pl.BlockSpec((1, BUCKET, 1), prev),
            pl.BlockSpec((1, BUCKET, 2 * DH), cur),
            pl.BlockSpec((1, BUCKET, 2 * DH), prev),
        ],
        out_specs=pl.BlockSpec((1, BUCKET, 2 * DH), cur),
        out_shape=jax.ShapeDtypeStruct((HEADS, S, 2 * DH), jnp.float32),
    )(sqkv, sqkv, nrm3, nrm3, stick, stick)


def _lsh_attention_block(qk, v, rot, tick_src):
    qkv = jnp.concatenate([qk[0], v[0]], axis=-1)
    r = _tc_hash(qk[0], rot)
    idx = _tc_rank(r).reshape(HEADS * S)
    sqkv, stick = _sc_scatter2(qkv.reshape(HEADS * S, 2 * DH),
                               tick_src, idx)
    sqkv3 = sqkv.reshape(HEADS, S, 2 * DH)
    nrm3 = jnp.linalg.norm(sqkv3[:, :, :DH], axis=-1, keepdims=True)
    so = _tc_attn(sqkv3, nrm3, stick.reshape(HEADS, S, 2 * DH))
    a = _sc_gather_rows(so.reshape(HEADS * S, 2 * DH), idx)
    return a.reshape(HEADS, S, 2 * DH)[None, :, :, :DH]


def _layer_norm(x, g, b):
    m = jnp.mean(x, axis=-1, keepdims=True)
    v = jnp.var(x, axis=-1, keepdims=True)
    return (x - m) / jnp.sqrt(v + 1e-5) * g + b


def kernel(source, tok_emb, proj_in, Wqk, Wv, Wo, ln1_g, ln1_b, ln2_g,
           ln2_b, Wff1, bff1, Wff2, bff2, lnf_g, lnf_b, Wout, bout):
    Bq = source.shape[0]
    depth = Wqk.shape[0]
    rot = _rot()
    tick_src = jnp.broadcast_to(
        (jnp.arange(HEADS * S, dtype=jnp.int32) % S
         ).astype(jnp.float32)[:, None], (HEADS * S, 2 * DH))

    src = source.reshape(Bq * S).astype(jnp.int32)
    emb_rows = _sc_gather_rows(tok_emb, src).reshape(Bq, S, EMB)
    x = emb_rows + _pos_emb()[None]
    x = x @ proj_in
    for l in range(depth):
        h = _layer_norm(x, ln1_g[l], ln1_b[l])
        qk = (h @ Wqk[l]).reshape(Bq, S, HEADS, DH).transpose(0, 2, 1, 3)
        v = (h @ Wv[l]).reshape(Bq, S, HEADS, DH).transpose(0, 2, 1, 3)
        a = _lsh_attention_block(qk, v, rot, tick_src)
        a = a.transpose(0, 2, 1, 3).reshape(Bq, S, DIM) @ Wo[l]
        x = x + a
        h = _layer_norm(x, ln2_g[l], ln2_b[l])
        x = x + (jax.nn.gelu(h @ Wff1[l] + bff1[l]) @ Wff2[l] + bff2[l])
    x = _layer_norm(x, lnf_g, lnf_b)
    logits = x @ Wout + bout
    return jax.nn.softmax(logits, axis=-1)
